# baseline (speedup 1.0000x reference)
import functools

import jax
import jax.numpy as jnp
from jax import lax
from jax.experimental import pallas as pl
from jax.experimental.pallas import tpu as pltpu
from jax.experimental.pallas import tpu_sc as plsc

N = 10000
E = 320000
D = 128

NT = 32
RPT = 320
NPAD = NT * RPT
SLICE = E // NT
BLK = 128
RLEN = 10240
NEG_INF = float("-inf")


def _bucket_body(src_hbm, dst_hbm, sl_hbm, dl_hbm, cnt_hbm,
                 sbuf, dbuf, sbb, dbb, offv, nblkv, cntv, sem):
    sid = lax.axis_index("s") * 2 + lax.axis_index("c")
    ca = pltpu.async_copy(src_hbm.at[pl.ds(sid * SLICE, SLICE)], sbuf, sem)
    cb = pltpu.async_copy(dst_hbm.at[pl.ds(sid * SLICE, SLICE)], dbuf, sem)
    ca.wait()
    cb.wait()

    zeros = jnp.zeros((16,), jnp.int32)
    for q in range(4):
        offv[pl.ds(q * 16, 16)] = zeros
        nblkv[pl.ds(q * 16, 16)] = zeros

    def group_body(j, _):
        liota = lax.iota(jnp.int32, 16)
        e0 = ((0 - liota) >> 31) + 1
        zer = jnp.zeros((16,), jnp.int32)
        d16 = dbuf[pl.ds(j * 16, 16)]
        s16 = sbuf[pl.ds(j * 16, 16)]
        for k in range(16):
            d = d16[k]
            s = s16[k]
            b = ((d >> 6) * 6554) >> 15
            l = d - b * 320
            ow = offv[pl.ds(b, 16)]
            ob = ow[0]
            addr = b * 256 + ob
            sbb[pl.ds(addr, 16)] = zer + s
            dbb[pl.ds(addr, 16)] = zer + l
            isf = (ob + 1) >> 7
            offv[pl.ds(b, 16)] = ow + e0 - (e0 * 128) * isf

            @pl.when(isf == 1)
            def _():
                nw = nblkv[pl.ds(b, 16)]
                nb = nw[0]
                row = sid * 32 + b
                pltpu.sync_copy(sbb.at[pl.ds(b * 256, BLK)],
                                sl_hbm.at[row, pl.ds(nb * BLK, BLK)])
                pltpu.sync_copy(dbb.at[pl.ds(b * 256, BLK)],
                                dl_hbm.at[row, pl.ds(nb * BLK, BLK)])
                nblkv[pl.ds(b, 16)] = nw + e0

        return 0

    lax.fori_loop(0, SLICE // 16, group_body, 0)

    zer = jnp.zeros((16,), jnp.int32)
    for b in range(32):
        ow = offv[pl.ds(b, 16)]
        ob = ow[0]
        for q in range(8):
            sbb[pl.ds(b * 256 + ob + q * 16, 16)] = zer
            dbb[pl.ds(b * 256 + ob + q * 16, 16)] = zer + RPT
        nw = nblkv[pl.ds(b, 16)]
        nb = nw[0]
        row = sid * 32 + b
        pltpu.sync_copy(sbb.at[pl.ds(b * 256, BLK)],
                        sl_hbm.at[row, pl.ds(nb * BLK, BLK)])
        pltpu.sync_copy(dbb.at[pl.ds(b * 256, BLK)],
                        dl_hbm.at[row, pl.ds(nb * BLK, BLK)])
        cntv[pl.ds(b, 16)] = zer + (nb * BLK + ob)

    pltpu.sync_copy(cntv.at[pl.ds(0, 64)], cnt_hbm.at[pl.ds(sid * 64, 64)])


def _bucket(src, dst):
    mesh = plsc.VectorSubcoreMesh(core_axis_name="c", subcore_axis_name="s")
    f = functools.partial(
        pl.kernel,
        mesh=mesh,
        out_type=[
            jax.ShapeDtypeStruct((NT * NT, RLEN), jnp.int32),
            jax.ShapeDtypeStruct((NT * NT, RLEN), jnp.int32),
            jax.ShapeDtypeStruct((NT * 64,), jnp.int32),
        ],
        scratch_types=[
            pltpu.VMEM((SLICE,), jnp.int32),
            pltpu.VMEM((SLICE,), jnp.int32),
            pltpu.VMEM((32 * 256,), jnp.int32),
            pltpu.VMEM((32 * 256,), jnp.int32),
            pltpu.VMEM((64,), jnp.int32),
            pltpu.VMEM((64,), jnp.int32),
            pltpu.VMEM((64,), jnp.int32),
            pltpu.SemaphoreType.DMA,
        ],
    )(_bucket_body)
    return f(src, dst)


def _segmax_body(h_hbm, sl_hbm, dl_hbm, cnt_hbm, out_hbm,
                 agg, cbuf, sblk, dblk, rows, sem):
    b = lax.axis_index("s") * 2 + lax.axis_index("c")

    def init_body(i, _):
        agg[pl.ds(i * 16, 16)] = jnp.full((16,), NEG_INF, jnp.float32)
        return 0
    lax.fori_loop(0, (RPT + 1) * D // 16, init_body, 0)

    pltpu.sync_copy(cnt_hbm, cbuf)

    def scan_body(s, _):
        cnt = cbuf[pl.ds(s * 64 + b, 16)][0]
        nb = (cnt + (BLK - 1)) >> 7
        row = s * 32 + b

        def blk_body(bb, _):
            ca = pltpu.async_copy(sl_hbm.at[row, pl.ds(bb * BLK, BLK)],
                                  sblk, sem)
            cc = pltpu.async_copy(dl_hbm.at[row, pl.ds(bb * BLK, BLK)],
                                  dblk, sem)
            ca.wait()
            cc.wait()
            g = pltpu.async_copy(h_hbm.at[sblk], rows, sem)
            g.wait()

            def acc_body(j2, _):
                dl16 = dblk[pl.ds(j2 * 16, 16)]
                for k in range(16):
                    dl = dl16[k]
                    base = dl * D
                    for c in range(8):
                        w = pl.ds(base + c * 16, 16)
                        rv = rows[j2 * 16 + k, pl.ds(c * 16, 16)]
                        agg[w] = jnp.maximum(agg[w], rv)
                return 0
            lax.fori_loop(0, BLK // 16, acc_body, 0)
            return 0

        lax.fori_loop(0, nb, blk_body, 0)
        return 0

    lax.fori_loop(0, 32, scan_body, 0)

    pltpu.sync_copy(agg.at[pl.ds(0, RPT * D)], out_hbm.at[b])


def _segmax(hp, slists, dlists, counts):
    mesh = plsc.VectorSubcoreMesh(core_axis_name="c", subcore_axis_name="s")
    f = functools.partial(
        pl.kernel,
        mesh=mesh,
        out_type=jax.ShapeDtypeStruct((NT, RPT * D), jnp.float32),
        scratch_types=[
            pltpu.VMEM(((RPT + 1) * D,), jnp.float32),
            pltpu.VMEM((NT * 64,), jnp.int32),
            pltpu.VMEM((BLK,), jnp.int32),
            pltpu.VMEM((BLK,), jnp.int32),
            pltpu.VMEM((BLK, D), jnp.float32),
            pltpu.SemaphoreType.DMA,
        ],
    )(_segmax_body)
    return f(hp, slists, dlists, counts)


def _affine_body(h_ref, agg_ref, w_ref, b_ref, eps_ref, o_ref, *, act):
    agg = agg_ref[...]
    agg = jnp.where(jnp.isfinite(agg), agg, 0.0)
    x = (1.0 + eps_ref[0]) * h_ref[...] + agg
    y = lax.dot_general(
        x, w_ref[...],
        dimension_numbers=(((1,), (1,)), ((), ())),
        preferred_element_type=jnp.float32,
    ) + b_ref[...]
    if act:
        y = jnp.where(y >= 0, y, 0.01 * y)
    o_ref[...] = y


def _affine(h, agg, W, b, eps, act):
    return pl.pallas_call(
        functools.partial(_affine_body, act=act),
        out_shape=jax.ShapeDtypeStruct((NPAD, D), jnp.float32),
        in_specs=[
            pl.BlockSpec(memory_space=pltpu.VMEM),
            pl.BlockSpec(memory_space=pltpu.VMEM),
            pl.BlockSpec(memory_space=pltpu.VMEM),
            pl.BlockSpec(memory_space=pltpu.VMEM),
            pl.BlockSpec(memory_space=pltpu.SMEM),
        ],
        out_specs=pl.BlockSpec(memory_space=pltpu.VMEM),
    )(h, agg, W, b.reshape(1, D), eps.reshape(1))


def kernel(n_feat, edge_index, W0, b0, eps0, W1, b1, eps1, W2, b2, eps2):
    src = edge_index[0]
    dst = edge_index[1]
    hp = jnp.pad(n_feat, ((0, NPAD - N), (0, 0)))
    slists, dlists, counts = _bucket(src, dst)
    params = ((W0, b0, eps0), (W1, b1, eps1), (W2, b2, eps2))
    for i, (Wt, b, eps) in enumerate(params):
        agg = _segmax(hp, slists, dlists, counts).reshape(NPAD, D)
        hp = _affine(hp, agg, Wt, b, eps, act=(i + 1 < len(params)))
    return hp[:N]


# speedup vs baseline: 1.0039x; 1.0039x over previous
import functools

import jax
import jax.numpy as jnp
from jax import lax
from jax.experimental import pallas as pl
from jax.experimental.pallas import tpu as pltpu
from jax.experimental.pallas import tpu_sc as plsc

N = 10000
E = 320000
D = 128

NT = 32
RPT = 320
NPAD = NT * RPT
SLICE = E // NT
BLK = 128
RLEN = 10240
NEG_INF = float("-inf")


def _bucket_body(src_hbm, dst_hbm, sl_hbm, dl_hbm, cnt_hbm,
                 sbuf, dbuf, sbb, dbb, offv, nblkv, cntv, sem):
    sid = lax.axis_index("s") * 2 + lax.axis_index("c")
    ca = pltpu.async_copy(src_hbm.at[pl.ds(sid * SLICE, SLICE)], sbuf, sem)
    cb = pltpu.async_copy(dst_hbm.at[pl.ds(sid * SLICE, SLICE)], dbuf, sem)
    ca.wait()
    cb.wait()

    zeros = jnp.zeros((16,), jnp.int32)
    for q in range(4):
        offv[pl.ds(q * 16, 16)] = zeros
        nblkv[pl.ds(q * 16, 16)] = zeros

    def group_body(j, _):
        liota = lax.iota(jnp.int32, 16)
        e0 = ((0 - liota) >> 31) + 1
        zer = jnp.zeros((16,), jnp.int32)
        d16 = dbuf[pl.ds(j * 16, 16)]
        s16 = sbuf[pl.ds(j * 16, 16)]
        for k in range(16):
            d = d16[k]
            s = s16[k]
            b = ((d >> 6) * 6554) >> 15
            l = d - b * 320
            ow = offv[pl.ds(b, 16)]
            ob = ow[0]
            addr = b * 256 + ob
            sbb[pl.ds(addr, 16)] = zer + s
            dbb[pl.ds(addr, 16)] = zer + l
            isf = (ob + 1) >> 7
            offv[pl.ds(b, 16)] = ow + e0 - (e0 * 128) * isf

            @pl.when(isf == 1)
            def _():
                nw = nblkv[pl.ds(b, 16)]
                nb = nw[0]
                row = sid * 32 + b
                pltpu.sync_copy(sbb.at[pl.ds(b * 256, BLK)],
                                sl_hbm.at[row, pl.ds(nb * BLK, BLK)])
                pltpu.sync_copy(dbb.at[pl.ds(b * 256, BLK)],
                                dl_hbm.at[row, pl.ds(nb * BLK, BLK)])
                nblkv[pl.ds(b, 16)] = nw + e0

        return 0

    lax.fori_loop(0, SLICE // 16, group_body, 0)

    zer = jnp.zeros((16,), jnp.int32)
    for b in range(32):
        ow = offv[pl.ds(b, 16)]
        ob = ow[0]
        for q in range(8):
            sbb[pl.ds(b * 256 + ob + q * 16, 16)] = zer
            dbb[pl.ds(b * 256 + ob + q * 16, 16)] = zer + RPT
        nw = nblkv[pl.ds(b, 16)]
        nb = nw[0]
        row = sid * 32 + b
        pltpu.sync_copy(sbb.at[pl.ds(b * 256, BLK)],
                        sl_hbm.at[row, pl.ds(nb * BLK, BLK)])
        pltpu.sync_copy(dbb.at[pl.ds(b * 256, BLK)],
                        dl_hbm.at[row, pl.ds(nb * BLK, BLK)])
        cntv[pl.ds(b, 16)] = zer + (nb * BLK + ob)

    pltpu.sync_copy(cntv.at[pl.ds(0, 64)], cnt_hbm.at[pl.ds(sid * 64, 64)])


def _bucket(src, dst):
    mesh = plsc.VectorSubcoreMesh(core_axis_name="c", subcore_axis_name="s")
    f = functools.partial(
        pl.kernel,
        mesh=mesh,
        out_type=[
            jax.ShapeDtypeStruct((NT * NT, RLEN), jnp.int32),
            jax.ShapeDtypeStruct((NT * NT, RLEN), jnp.int32),
            jax.ShapeDtypeStruct((NT * 64,), jnp.int32),
        ],
        scratch_types=[
            pltpu.VMEM((SLICE,), jnp.int32),
            pltpu.VMEM((SLICE,), jnp.int32),
            pltpu.VMEM((32 * 256,), jnp.int32),
            pltpu.VMEM((32 * 256,), jnp.int32),
            pltpu.VMEM((64,), jnp.int32),
            pltpu.VMEM((64,), jnp.int32),
            pltpu.VMEM((64,), jnp.int32),
            pltpu.SemaphoreType.DMA,
        ],
    )(_bucket_body)
    return f(src, dst)


def _segmax_body(h_hbm, sl_hbm, dl_hbm, cnt_hbm, out_hbm,
                 agg, cbuf, sblk, dblk, rows, sem):
    b = lax.axis_index("s") * 2 + lax.axis_index("c")

    def init_body(i, _):
        for c in range(8):
            agg[i, pl.ds(c * 16, 16)] = jnp.full((16,), NEG_INF, jnp.float32)
        return 0
    lax.fori_loop(0, RPT + 1, init_body, 0)

    pltpu.sync_copy(cnt_hbm, cbuf)

    def scan_body(s, _):
        cnt = cbuf[pl.ds(s * 64 + b, 16)][0]
        nb = (cnt + (BLK - 1)) >> 7
        row = s * 32 + b

        def blk_body(bb, _):
            ca = pltpu.async_copy(sl_hbm.at[row, pl.ds(bb * BLK, BLK)],
                                  sblk, sem)
            cc = pltpu.async_copy(dl_hbm.at[row, pl.ds(bb * BLK, BLK)],
                                  dblk, sem)
            ca.wait()
            cc.wait()
            g = pltpu.async_copy(h_hbm.at[sblk], rows, sem)
            g.wait()

            def acc_body(j2, _):
                dl16 = dblk[pl.ds(j2 * 16, 16)]
                for k in range(16):
                    dl = dl16[k]
                    for c in range(8):
                        w = pl.ds(c * 16, 16)
                        rv = rows[j2 * 16 + k, w]
                        agg[dl, w] = jnp.maximum(agg[dl, w], rv)
                return 0
            lax.fori_loop(0, BLK // 16, acc_body, 0)
            return 0

        lax.fori_loop(0, nb, blk_body, 0)
        return 0

    lax.fori_loop(0, 32, scan_body, 0)

    pltpu.sync_copy(agg.at[pl.ds(0, RPT)], out_hbm.at[b])


def _segmax(hp, slists, dlists, counts):
    mesh = plsc.VectorSubcoreMesh(core_axis_name="c", subcore_axis_name="s")
    f = functools.partial(
        pl.kernel,
        mesh=mesh,
        out_type=jax.ShapeDtypeStruct((NT, RPT, D), jnp.float32),
        scratch_types=[
            pltpu.VMEM((RPT + 1, D), jnp.float32),
            pltpu.VMEM((NT * 64,), jnp.int32),
            pltpu.VMEM((BLK,), jnp.int32),
            pltpu.VMEM((BLK,), jnp.int32),
            pltpu.VMEM((BLK, D), jnp.float32),
            pltpu.SemaphoreType.DMA,
        ],
    )(_segmax_body)
    return f(hp, slists, dlists, counts)


def _affine_body(h_ref, agg_ref, w_ref, b_ref, eps_ref, o_ref, *, act):
    agg = agg_ref[...]
    agg = jnp.where(jnp.isfinite(agg), agg, 0.0)
    x = (1.0 + eps_ref[0]) * h_ref[...] + agg
    y = lax.dot_general(
        x, w_ref[...],
        dimension_numbers=(((1,), (1,)), ((), ())),
        preferred_element_type=jnp.float32,
    ) + b_ref[...]
    if act:
        y = jnp.where(y >= 0, y, 0.01 * y)
    o_ref[...] = y


def _affine(h, agg, W, b, eps, act):
    return pl.pallas_call(
        functools.partial(_affine_body, act=act),
        out_shape=jax.ShapeDtypeStruct((NPAD, D), jnp.float32),
        in_specs=[
            pl.BlockSpec(memory_space=pltpu.VMEM),
            pl.BlockSpec(memory_space=pltpu.VMEM),
            pl.BlockSpec(memory_space=pltpu.VMEM),
            pl.BlockSpec(memory_space=pltpu.VMEM),
            pl.BlockSpec(memory_space=pltpu.SMEM),
        ],
        out_specs=pl.BlockSpec(memory_space=pltpu.VMEM),
    )(h, agg, W, b.reshape(1, D), eps.reshape(1))


def kernel(n_feat, edge_index, W0, b0, eps0, W1, b1, eps1, W2, b2, eps2):
    src = edge_index[0]
    dst = edge_index[1]
    hp = jnp.pad(n_feat, ((0, NPAD - N), (0, 0)))
    slists, dlists, counts = _bucket(src, dst)
    params = ((W0, b0, eps0), (W1, b1, eps1), (W2, b2, eps2))
    for i, (Wt, b, eps) in enumerate(params):
        agg = _segmax(hp, slists, dlists, counts).reshape(NPAD, D)
        hp = _affine(hp, agg, Wt, b, eps, act=(i + 1 < len(params)))
    return hp[:N]


# speedup vs baseline: 1.0077x; 1.0037x over previous
import functools

import jax
import jax.numpy as jnp
from jax import lax
from jax.experimental import pallas as pl
from jax.experimental.pallas import tpu as pltpu
from jax.experimental.pallas import tpu_sc as plsc

N = 10000
E = 320000
D = 128

NT = 32
RPT = 320
NPAD = NT * RPT
SLICE = E // NT
BLK = 128
RLEN = 10240
NEG_INF = float("-inf")


def _bucket_body(src_hbm, dst_hbm, sl_hbm, dl_hbm, cnt_hbm,
                 sbuf, dbuf, sbb, dbb, offv, nblkv, cntv, sem):
    sid = lax.axis_index("s") * 2 + lax.axis_index("c")
    ca = pltpu.async_copy(src_hbm.at[pl.ds(sid * SLICE, SLICE)], sbuf, sem)
    cb = pltpu.async_copy(dst_hbm.at[pl.ds(sid * SLICE, SLICE)], dbuf, sem)
    ca.wait()
    cb.wait()

    zeros = jnp.zeros((16,), jnp.int32)
    for q in range(4):
        offv[pl.ds(q * 16, 16)] = zeros
        nblkv[pl.ds(q * 16, 16)] = zeros

    def group_body(j, _):
        liota = lax.iota(jnp.int32, 16)
        e0 = ((0 - liota) >> 31) + 1
        zer = jnp.zeros((16,), jnp.int32)
        d16 = dbuf[pl.ds(j * 16, 16)]
        s16 = sbuf[pl.ds(j * 16, 16)]
        for k in range(16):
            d = d16[k]
            s = s16[k]
            b = ((d >> 6) * 6554) >> 15
            l = d - b * 320
            ow = offv[pl.ds(b, 16)]
            ob = ow[0]
            addr = b * 256 + ob
            sbb[pl.ds(addr, 16)] = zer + s
            dbb[pl.ds(addr, 16)] = zer + l
            isf = (ob + 1) >> 7
            offv[pl.ds(b, 16)] = ow + e0 - (e0 * 128) * isf

            @pl.when(isf == 1)
            def _():
                nw = nblkv[pl.ds(b, 16)]
                nb = nw[0]
                row = sid * 32 + b
                pltpu.sync_copy(sbb.at[pl.ds(b * 256, BLK)],
                                sl_hbm.at[row, pl.ds(nb * BLK, BLK)])
                pltpu.sync_copy(dbb.at[pl.ds(b * 256, BLK)],
                                dl_hbm.at[row, pl.ds(nb * BLK, BLK)])
                nblkv[pl.ds(b, 16)] = nw + e0

        return 0

    lax.fori_loop(0, SLICE // 16, group_body, 0)

    zer = jnp.zeros((16,), jnp.int32)
    for b in range(32):
        ow = offv[pl.ds(b, 16)]
        ob = ow[0]
        for q in range(8):
            sbb[pl.ds(b * 256 + ob + q * 16, 16)] = zer
            dbb[pl.ds(b * 256 + ob + q * 16, 16)] = zer + RPT
        nw = nblkv[pl.ds(b, 16)]
        nb = nw[0]
        row = sid * 32 + b
        pltpu.sync_copy(sbb.at[pl.ds(b * 256, BLK)],
                        sl_hbm.at[row, pl.ds(nb * BLK, BLK)])
        pltpu.sync_copy(dbb.at[pl.ds(b * 256, BLK)],
                        dl_hbm.at[row, pl.ds(nb * BLK, BLK)])
        cntv[pl.ds(b, 16)] = zer + (nb * BLK + ob)

    pltpu.sync_copy(cntv.at[pl.ds(0, 64)], cnt_hbm.at[pl.ds(sid * 64, 64)])


def _bucket(src, dst):
    mesh = plsc.VectorSubcoreMesh(core_axis_name="c", subcore_axis_name="s")
    f = functools.partial(
        pl.kernel,
        mesh=mesh,
        out_type=[
            jax.ShapeDtypeStruct((NT * NT, RLEN), jnp.int32),
            jax.ShapeDtypeStruct((NT * NT, RLEN), jnp.int32),
            jax.ShapeDtypeStruct((NT * 64,), jnp.int32),
        ],
        scratch_types=[
            pltpu.VMEM((SLICE,), jnp.int32),
            pltpu.VMEM((SLICE,), jnp.int32),
            pltpu.VMEM((32 * 256,), jnp.int32),
            pltpu.VMEM((32 * 256,), jnp.int32),
            pltpu.VMEM((64,), jnp.int32),
            pltpu.VMEM((64,), jnp.int32),
            pltpu.VMEM((64,), jnp.int32),
            pltpu.SemaphoreType.DMA,
        ],
    )(_bucket_body)
    return f(src, dst)


def _segmax_body(h_hbm, sl_hbm, dl_hbm, cnt_hbm, out_hbm,
                 agg, cbuf, sblk, dblk, rows, sem):
    b = lax.axis_index("s") * 2 + lax.axis_index("c")

    def init_body(i, _):
        for c in range(8):
            agg[i, pl.ds(c * 16, 16)] = jnp.full((16,), NEG_INF, jnp.float32)
        return 0
    lax.fori_loop(0, RPT + 1, init_body, 0)

    pltpu.sync_copy(cnt_hbm, cbuf)

    def scan_body(s, _):
        cnt = cbuf[pl.ds(s * 64 + b, 16)][0]
        nb = (cnt + (BLK - 1)) >> 7
        row = s * 32 + b

        def blk_body(bb, _):
            ca = pltpu.async_copy(sl_hbm.at[row, pl.ds(bb * BLK, BLK)],
                                  sblk, sem)
            cc = pltpu.async_copy(dl_hbm.at[row, pl.ds(bb * BLK, BLK)],
                                  dblk, sem)
            ca.wait()
            cc.wait()
            g = pltpu.async_copy(h_hbm.at[sblk], rows, sem)
            g.wait()

            def acc_body(j2, _):
                dl16 = dblk[pl.ds(j2 * 16, 16)]
                for k in range(16):
                    dl = dl16[k]
                    mx = [jnp.maximum(agg[dl, pl.ds(c * 16, 16)],
                                      rows[j2 * 16 + k, pl.ds(c * 16, 16)])
                          for c in range(8)]
                    for c in range(8):
                        agg[dl, pl.ds(c * 16, 16)] = mx[c]
                return 0
            lax.fori_loop(0, BLK // 16, acc_body, 0)
            return 0

        lax.fori_loop(0, nb, blk_body, 0)
        return 0

    lax.fori_loop(0, 32, scan_body, 0)

    pltpu.sync_copy(agg.at[pl.ds(0, RPT)], out_hbm.at[b])


def _segmax(hp, slists, dlists, counts):
    mesh = plsc.VectorSubcoreMesh(core_axis_name="c", subcore_axis_name="s")
    f = functools.partial(
        pl.kernel,
        mesh=mesh,
        out_type=jax.ShapeDtypeStruct((NT, RPT, D), jnp.float32),
        scratch_types=[
            pltpu.VMEM((RPT + 1, D), jnp.float32),
            pltpu.VMEM((NT * 64,), jnp.int32),
            pltpu.VMEM((BLK,), jnp.int32),
            pltpu.VMEM((BLK,), jnp.int32),
            pltpu.VMEM((BLK, D), jnp.float32),
            pltpu.SemaphoreType.DMA,
        ],
    )(_segmax_body)
    return f(hp, slists, dlists, counts)


def _affine_body(h_ref, agg_ref, w_ref, b_ref, eps_ref, o_ref, *, act):
    agg = agg_ref[...]
    agg = jnp.where(jnp.isfinite(agg), agg, 0.0)
    x = (1.0 + eps_ref[0]) * h_ref[...] + agg
    y = lax.dot_general(
        x, w_ref[...],
        dimension_numbers=(((1,), (1,)), ((), ())),
        preferred_element_type=jnp.float32,
    ) + b_ref[...]
    if act:
        y = jnp.where(y >= 0, y, 0.01 * y)
    o_ref[...] = y


def _affine(h, agg, W, b, eps, act):
    return pl.pallas_call(
        functools.partial(_affine_body, act=act),
        out_shape=jax.ShapeDtypeStruct((NPAD, D), jnp.float32),
        in_specs=[
            pl.BlockSpec(memory_space=pltpu.VMEM),
            pl.BlockSpec(memory_space=pltpu.VMEM),
            pl.BlockSpec(memory_space=pltpu.VMEM),
            pl.BlockSpec(memory_space=pltpu.VMEM),
            pl.BlockSpec(memory_space=pltpu.SMEM),
        ],
        out_specs=pl.BlockSpec(memory_space=pltpu.VMEM),
    )(h, agg, W, b.reshape(1, D), eps.reshape(1))


def kernel(n_feat, edge_index, W0, b0, eps0, W1, b1, eps1, W2, b2, eps2):
    src = edge_index[0]
    dst = edge_index[1]
    hp = jnp.pad(n_feat, ((0, NPAD - N), (0, 0)))
    slists, dlists, counts = _bucket(src, dst)
    params = ((W0, b0, eps0), (W1, b1, eps1), (W2, b2, eps2))
    for i, (Wt, b, eps) in enumerate(params):
        agg = _segmax(hp, slists, dlists, counts).reshape(NPAD, D)
        hp = _affine(hp, agg, Wt, b, eps, act=(i + 1 < len(params)))
    return hp[:N]
